# baseline (speedup 1.0000x reference)
import jax
import jax.numpy as jnp
from jax import lax
from jax.experimental import pallas as pl
from jax.experimental.pallas import tpu as pltpu
from jax.experimental.pallas import tpu_sc as plsc

SEQ_LEN = 512
CLS_ID = 101
SEP_ID = 102
PAD_ID = 0
BATCH = 16
TOTAL_TOKENS = 32768

_L = 16
_NC = 2
_NS = 16
_NW = _NC * _NS
_CHUNK = (BATCH * SEQ_LEN) // _NW


def _body(s_hbm, e_hbm, flat_hbm, out_hbm, mask_hbm, s_v, e_v, idx_v, gath_v, out_v, mask_v, sem):
    wid = lax.axis_index("s") * _NC + lax.axis_index("c")
    h = wid % 2
    base = h * _CHUNK

    pltpu.sync_copy(s_hbm.at[wid], s_v)
    pltpu.sync_copy(e_hbm.at[wid], e_v)
    sv = s_v[...]
    ev = e_v[...]
    tv = jnp.minimum(ev - sv, SEQ_LEN - 2)

    for r in range(2):
        for j in range(8):
            pos = base + r * 128 + j * _L + lax.iota(jnp.int32, _L)
            g = jnp.clip(sv + pos - 1, 0, TOTAL_TOKENS - 1)
            idx_v[pl.ds(j * _L, _L)] = g
        pltpu.async_copy(flat_hbm.at[idx_v], gath_v.at[pl.ds(r * 128, 128)], sem).wait()

    for j in range(_CHUNK // _L):
        pos = base + j * _L + lax.iota(jnp.int32, _L)
        tok = gath_v[pl.ds(j * _L, _L)]
        out = jnp.where(
            pos == 0,
            jnp.int32(CLS_ID),
            jnp.where(
                pos <= tv,
                tok,
                jnp.where(pos == tv + 1, jnp.int32(SEP_ID), jnp.int32(PAD_ID)),
            ),
        )
        out_v[pl.ds(j * _L, _L)] = out
        mask_v[pl.ds(j * _L, _L)] = jnp.where(out == PAD_ID, jnp.int32(0), jnp.int32(1))

    pltpu.sync_copy(out_v, out_hbm.at[wid])
    pltpu.sync_copy(mask_v, mask_hbm.at[wid])


def kernel(flat_tokens, cu_seqlens):
    cu = cu_seqlens.astype(jnp.int32)
    s_splat = jnp.broadcast_to(jnp.repeat(cu[:-1], 2)[:, None], (_NW, _L))
    e_splat = jnp.broadcast_to(jnp.repeat(cu[1:], 2)[:, None], (_NW, _L))
    mesh = plsc.VectorSubcoreMesh(core_axis_name="c", subcore_axis_name="s")
    packed, mask = pl.kernel(
        _body,
        mesh=mesh,
        out_type=[
            jax.ShapeDtypeStruct((_NW, _CHUNK), jnp.int32),
            jax.ShapeDtypeStruct((_NW, _CHUNK), jnp.int32),
        ],
        scratch_types=[
            pltpu.VMEM((_L,), jnp.int32),
            pltpu.VMEM((_L,), jnp.int32),
            pltpu.VMEM((128,), jnp.int32),
            pltpu.VMEM((_CHUNK,), jnp.int32),
            pltpu.VMEM((_CHUNK,), jnp.int32),
            pltpu.VMEM((_CHUNK,), jnp.int32),
            pltpu.SemaphoreType.DMA,
        ],
    )(s_splat, e_splat, flat_tokens.astype(jnp.int32))
    token_ids = packed.reshape(BATCH, SEQ_LEN)
    padding_mask = mask.reshape(BATCH, SEQ_LEN).astype(jnp.bool_)
    return token_ids, padding_mask


# speedup vs baseline: 1.1122x; 1.1122x over previous
import jax
import jax.numpy as jnp
from jax import lax
from jax.experimental import pallas as pl
from jax.experimental.pallas import tpu as pltpu
from jax.experimental.pallas import tpu_sc as plsc

SEQ_LEN = 512
CLS_ID = 101
SEP_ID = 102
PAD_ID = 0
BATCH = 16
TOTAL_TOKENS = 32768

_L = 16
_NC = 2
_NS = 16
_NW = _NC * _NS
_CHUNK = (BATCH * SEQ_LEN) // _NW
_WIN = 288
_WBUF = 832


def _body(cu_hbm, flat_hbm, out_hbm, mask_hbm, cu_v, win_v, out_v, mask_v, sem):
    wid = lax.axis_index("s") * _NC + lax.axis_index("c")
    b = wid // 2
    h = wid % 2
    base = h * _CHUNK

    pltpu.sync_copy(cu_hbm.at[pl.ds(0, _L)], cu_v)
    iv = lax.iota(jnp.int32, _L)
    v0 = cu_v[...]
    pick_idx = jnp.minimum(
        jnp.full((_L,), b, jnp.int32) + jnp.where(iv == 0, 0, 1), _L - 1
    )
    picked = lax.gather(
        v0,
        pick_idx[:, None],
        lax.GatherDimensionNumbers(
            offset_dims=(), collapsed_slice_dims=(0,), start_index_map=(0,)
        ),
        slice_sizes=(1,),
        mode=lax.GatherScatterMode.PROMISE_IN_BOUNDS,
    )
    s = picked[0]
    e_lo = picked[1]
    e = jnp.where(b == BATCH - 1, jnp.int32(TOTAL_TOKENS), e_lo)
    t = jnp.minimum(e - s, SEQ_LEN - 2)

    aoff = jnp.clip(s + base - 8, 0, TOTAL_TOKENS - _WIN) & ~jnp.int32(7)
    aoff = pl.multiple_of(aoff, 8)
    d = s + base - 1 - aoff
    pltpu.async_copy(
        flat_hbm.at[pl.ds(aoff, _WIN)], win_v.at[pl.ds(8, _WIN)], sem
    ).wait()

    for j in range(_CHUNK // _L):
        pos = base + j * _L + iv
        tok = win_v[pl.ds(8 + d + j * _L, _L)]
        out = jnp.where(
            pos == 0,
            jnp.int32(CLS_ID),
            jnp.where(
                pos <= t,
                tok,
                jnp.where(pos == t + 1, jnp.int32(SEP_ID), jnp.int32(PAD_ID)),
            ),
        )
        out_v[pl.ds(j * _L, _L)] = out
        mask_v[pl.ds(j * _L, _L)] = jnp.where(out == PAD_ID, jnp.int32(0), jnp.int32(1))

    c1 = pltpu.async_copy(out_v, out_hbm.at[wid], sem)
    c2 = pltpu.async_copy(mask_v, mask_hbm.at[wid], sem)
    c1.wait()
    c2.wait()


def kernel(flat_tokens, cu_seqlens):
    mesh = plsc.VectorSubcoreMesh(core_axis_name="c", subcore_axis_name="s")
    packed, mask = pl.kernel(
        _body,
        mesh=mesh,
        out_type=[
            jax.ShapeDtypeStruct((_NW, _CHUNK), jnp.int32),
            jax.ShapeDtypeStruct((_NW, _CHUNK), jnp.int32),
        ],
        scratch_types=[
            pltpu.VMEM((_L,), jnp.int32),
            pltpu.VMEM((_WBUF,), jnp.int32),
            pltpu.VMEM((_CHUNK,), jnp.int32),
            pltpu.VMEM((_CHUNK,), jnp.int32),
            pltpu.SemaphoreType.DMA,
        ],
    )(cu_seqlens.astype(jnp.int32), flat_tokens.astype(jnp.int32))
    token_ids = packed.reshape(BATCH, SEQ_LEN)
    padding_mask = mask.reshape(BATCH, SEQ_LEN).astype(jnp.bool_)
    return token_ids, padding_mask
